# Initial kernel scaffold; baseline (speedup 1.0000x reference)
import jax
import jax.numpy as jnp
from jax.experimental import pallas as pl


def kernel(y, grid, inc):
    raise NotImplementedError("write your pallas kernel here")


# speedup vs baseline: 298.1942x; 298.1942x over previous
import functools

import jax
import jax.numpy as jnp
from jax import lax
from jax.experimental import pallas as pl
from jax.experimental.pallas import tpu as pltpu
from jax.experimental.pallas import tpu_sc as plsc

NC = 2
NS = 16
NW = NC * NS
L = 16

CHUNK = 256


@functools.partial(jax.jit, static_argnames=("ninc",))
def _sc_vegas(y, grid, inc, linc, *, ninc):
    B, D = y.shape
    assert D == 2 * L
    rows_per_w = B // NW
    n_chunks = rows_per_w // CHUNK
    assert rows_per_w % CHUNK == 0

    mesh = plsc.VectorSubcoreMesh(core_axis_name="c", subcore_axis_name="s")

    @functools.partial(
        pl.kernel,
        out_type=(
            jax.ShapeDtypeStruct((B, D), jnp.float32),
            jax.ShapeDtypeStruct((B,), jnp.float32),
        ),
        mesh=mesh,
        compiler_params=pltpu.CompilerParams(
            use_tc_tiling_on_sc=False, needs_layout_passes=False
        ),
        scratch_types=[
            pltpu.VMEM((D, ninc + 1), jnp.float32),
            pltpu.VMEM((D, ninc), jnp.float32),
            pltpu.VMEM((D, ninc), jnp.float32),
            pltpu.VMEM((CHUNK, D), jnp.float32),
            pltpu.VMEM((CHUNK, D), jnp.float32),
            pltpu.VMEM((CHUNK,), jnp.float32),
        ],
    )
    def k(y_hbm, grid_hbm, inc_hbm, linc_hbm, x_hbm, lj_hbm,
          grid_v, inc_v, linc_v, y_v, x_v, lj_v):
        cid = lax.axis_index("c")
        sid = lax.axis_index("s")
        wid = sid * NC + cid
        base = wid * rows_per_w

        pltpu.sync_copy(grid_hbm, grid_v)
        pltpu.sync_copy(inc_hbm, inc_v)
        pltpu.sync_copy(linc_hbm, linc_v)

        dvec0 = lax.iota(jnp.int32, L)
        dvec1 = dvec0 + L
        last_lane = dvec0 == (L - 1)
        ninc_f = jnp.float32(ninc)

        @pl.loop(0, n_chunks)
        def _chunk(ci):
            row0 = base + ci * CHUNK
            pltpu.sync_copy(y_hbm.at[pl.ds(row0, CHUNK)], y_v)

            @pl.loop(0, CHUNK)
            def _row(r):
                lj_acc = None
                for h, dvec in ((0, dvec0), (1, dvec1)):
                    yv = y_v[r, pl.ds(h * L, L)]
                    t = yv * ninc_f
                    iy = t.astype(jnp.int32)
                    dy = t - iy.astype(jnp.float32)
                    iy = jnp.maximum(iy, 0)
                    iy_g = jnp.minimum(iy, ninc)
                    iy_i = jnp.minimum(iy, ninc - 1)
                    g = plsc.load_gather(grid_v, [dvec, iy_g])
                    ic = plsc.load_gather(inc_v, [dvec, iy_i])
                    lg = plsc.load_gather(linc_v, [dvec, iy_i])
                    x_v[r, pl.ds(h * L, L)] = g + ic * dy
                    lj_acc = lg if lj_acc is None else lj_acc + lg
                tot = plsc.cumsum(lj_acc)
                ridx = jnp.full((L,), r, dtype=jnp.int32)
                plsc.store_scatter(lj_v, [ridx], tot, mask=last_lane)

            pltpu.sync_copy(x_v, x_hbm.at[pl.ds(row0, CHUNK)])
            pltpu.sync_copy(lj_v, lj_hbm.at[pl.ds(row0, CHUNK)])

    return k(y, grid, inc, linc)


def kernel(y, grid, inc):
    ninc = inc.shape[1]
    linc = jnp.log(inc * jnp.float32(ninc))
    return _sc_vegas(y, grid, inc, linc, ninc=ninc)
